# baseline (speedup 1.0000x reference)
import functools
import math

import jax
import jax.numpy as jnp
from jax import lax
from jax.experimental import pallas as pl
from jax.experimental.pallas import tpu as pltpu
from jax.experimental.pallas import tpu_sc as plsc

_EMBED_DIM = 32
_SEQ_LEN = 200
_BATCH = 4096
_NW = 32
_SEQ_PER_W = _BATCH // _NW
_CSEQ = 8
_N_CHUNKS = _SEQ_PER_W // _CSEQ
_SCALE = math.sqrt(_EMBED_DIM)
_H = _EMBED_DIM // 2


@jax.jit
def _tok_embed(x, table, pe):
    mesh = plsc.VectorSubcoreMesh(core_axis_name="c", subcore_axis_name="s")

    @functools.partial(
        pl.kernel,
        mesh=mesh,
        compiler_params=pltpu.CompilerParams(use_tc_tiling_on_sc=False),
        out_type=jax.ShapeDtypeStruct((_BATCH, _SEQ_LEN, _EMBED_DIM), jnp.float32),
        scratch_types=[
            pltpu.VMEM((_CSEQ, _SEQ_LEN), jnp.int32),
            pltpu.VMEM((_CSEQ, _SEQ_LEN, _EMBED_DIM), jnp.float32),
            pltpu.VMEM((_SEQ_LEN, _EMBED_DIM), jnp.float32),
            pltpu.SemaphoreType.DMA,
        ],
    )
    def k(x_hbm, table_hbm, pe_hbm, out_hbm, idx_v, rows_v, pe_v, sem):
        wid = lax.axis_index("s") * 2 + lax.axis_index("c")
        seq_base = wid * _SEQ_PER_W
        pltpu.sync_copy(pe_hbm.at[0, pl.ds(0, _SEQ_LEN), :], pe_v)

        def chunk_body(g, carry):
            s0 = seq_base + g * _CSEQ
            pltpu.sync_copy(x_hbm.at[pl.ds(s0, _CSEQ), :], idx_v)
            descs = [
                pltpu.async_copy(table_hbm.at[idx_v.at[j]], rows_v.at[j], sem)
                for j in range(_CSEQ)
            ]
            for d in descs:
                d.wait()

            def p_body(p, c2):
                pe_lo = pe_v[p, pl.ds(0, _H)]
                pe_hi = pe_v[p, pl.ds(_H, _H)]
                for j in range(_CSEQ):
                    rows_v[j, p, pl.ds(0, _H)] = (
                        rows_v[j, p, pl.ds(0, _H)] * _SCALE + pe_lo
                    )
                    rows_v[j, p, pl.ds(_H, _H)] = (
                        rows_v[j, p, pl.ds(_H, _H)] * _SCALE + pe_hi
                    )
                return c2

            lax.fori_loop(0, _SEQ_LEN, p_body, carry)
            pltpu.sync_copy(rows_v, out_hbm.at[pl.ds(s0, _CSEQ), :, :])
            return carry

        lax.fori_loop(0, _N_CHUNKS, chunk_body, 0)

    return k(x, table, pe)


def kernel(x, table, pe):
    return _tok_embed(x, table, pe)
